# Initial kernel scaffold; baseline (speedup 1.0000x reference)
import jax
import jax.numpy as jnp
from jax.experimental import pallas as pl


def kernel(hidden_states, in_proj_w, conv_w, conv_b, conv_w_b, conv_b_b, x_proj_w, x_proj_w_bwd, dt_proj_w, dt_proj_bias, dt_proj_w_bwd, dt_proj_bias_bwd, A_log, A_b_log, D, D_b, mlp_w1, mlp_b1, mlp_w2, mlp_b2, out_proj_w):
    raise NotImplementedError("write your pallas kernel here")


# speedup vs baseline: 8.0578x; 8.0578x over previous
import jax
import jax.numpy as jnp
from jax import lax
from jax.experimental import pallas as pl
from jax.experimental.pallas import tpu as pltpu

DM = 768
DS = 16
DC = 4
DI = 1536
NE = 4
DE = 4
R = 48
B = 4
L = 512
TC = 128
F32 = jnp.float32
_VMEM_LIM = 64 * 1024 * 1024


def _dot_nt(a, b):
    return lax.dot_general(a, b, (((1,), (1,)), ((), ())),
                           preferred_element_type=F32)


def _k1_body(hid_ref, win_ref, convw_ref, convb_ref, wdt_ref, wb_ref, wc_ref,
             dtp_ref, dtb_ref, w1_ref, b1_ref, w2_ref, b2_ref,
             xt_ref, gz_ref, delta_ref, bmt_ref, cmt_ref, wt_ref):
    i = pl.program_id(0)
    hid = hid_ref[0]
    xz = _dot_nt(hid, win_ref[...])
    x = xz[:, :DI]
    z = xz[:, DI:]

    wt = convw_ref[0]
    acc = x * wt[DC - 1:DC, :]
    for k in range(DC - 1):
        shift = DC - 1 - k
        xk = jnp.concatenate(
            [jnp.zeros((shift, DI), F32), x[:L - shift, :]], axis=0)
        acc = acc + xk * wt[k:k + 1, :]
    acc = acc + convb_ref[0]
    xc = acc * jax.nn.sigmoid(acc)

    dtr = _dot_nt(xc, wdt_ref[0])
    bmt = _dot_nt(wb_ref[0], xc)
    cmt = _dot_nt(wc_ref[0], xc)
    dv = _dot_nt(dtr, dtp_ref[0]) + dtb_ref[0]
    delta = jnp.maximum(dv, 0.0) + jnp.log1p(jnp.exp(-jnp.abs(dv)))

    xt_ref[0] = xc
    gz_ref[0] = z * jax.nn.sigmoid(z)
    delta_ref[0] = delta
    bmt_ref[0] = bmt
    cmt_ref[0] = cmt

    @pl.when(i % 2 == 0)
    def _router():
        h1 = _dot_nt(xc, w1_ref[...]) + b1_ref[0]
        h1 = h1 * jax.nn.sigmoid(h1)
        gt = _dot_nt(w2_ref[...], h1) + b2_ref[...]
        gt = jax.nn.sigmoid(gt)
        m = jnp.max(gt, axis=0, keepdims=True)
        e = jnp.exp(gt - m)
        wt_ref[0] = e / jnp.sum(e, axis=0, keepdims=True)


def _k2_body(delta_ref, xt_ref, gz_ref, bmt_ref, cmt_ref, wt_ref,
             alogt_ref, d_ref, y_ref):
    a_t = -jnp.exp(alogt_ref[0])
    drow_skip = d_ref[0]

    def chunk(c, h):
        base = c * TC
        sl = pl.ds(base, TC)
        delta_c = delta_ref[0, sl, :]
        xt_c = xt_ref[0, sl, :]
        gz_c = gz_ref[0, sl, :]
        bmt_c = bmt_ref[0, :, sl]
        cmt_c = cmt_ref[0, :, sl]
        wt_c = wt_ref[0, :, sl]
        wexp = jnp.concatenate(
            [wt_c[e:e + 1, :] for e in range(NE) for _ in range(DE)], axis=0)
        cwt = cmt_c * wexp
        rows = []
        for t in range(TC):
            drow = delta_c[t:t + 1, :]
            dxrow = drow * xt_c[t:t + 1, :]
            da = jnp.exp(a_t * drow)
            h = da * h + dxrow * bmt_c[:, t:t + 1]
            rows.append(jnp.sum(h * cwt[:, t:t + 1], axis=0, keepdims=True))
        y_c = jnp.concatenate(rows, axis=0)
        y_ref[0, sl, :] = (y_c + xt_c * drow_skip) * gz_c
        return h

    lax.fori_loop(0, L // TC, chunk, jnp.zeros((DS, DI), F32))


def _k3_body(yf_ref, yb_ref, wout_ref, out_ref):
    out_ref[0] = _dot_nt(yf_ref[0] + yb_ref[0], wout_ref[...])


def kernel(hidden_states, in_proj_w, conv_w, conv_b, conv_w_b, conv_b_b,
           x_proj_w, x_proj_w_bwd, dt_proj_w, dt_proj_bias, dt_proj_w_bwd,
           dt_proj_bias_bwd, A_log, A_b_log, D, D_b, mlp_w1, mlp_b1,
           mlp_w2, mlp_b2, out_proj_w):
    f32 = lambda v: v.astype(F32)
    hid2 = jnp.stack([hidden_states, hidden_states[:, ::-1, :]],
                     axis=1).reshape(2 * B, L, DM)
    convw2 = jnp.stack([conv_w[:, 0, :].T, conv_w_b[:, 0, :].T])
    convb2 = jnp.stack([conv_b, conv_b_b])[:, None, :]
    wdt2 = jnp.stack([x_proj_w[:R], x_proj_w_bwd[:R]])
    wb2 = jnp.stack([x_proj_w[R:R + DS], x_proj_w_bwd[R:R + DS]])
    wc2 = jnp.stack([x_proj_w[R + DS:], x_proj_w_bwd[R + DS:]])
    dtp2 = jnp.stack([dt_proj_w, dt_proj_w_bwd])
    dtb2 = jnp.stack([dt_proj_bias, dt_proj_bias_bwd])[:, None, :]
    alogt2 = jnp.stack([A_log.T, A_b_log.T])
    d2 = jnp.stack([D, D_b])[:, None, :]
    b1r = mlp_b1[None, None, :]
    b2r = mlp_b2[:, None]

    big = lambda n: pl.BlockSpec((1, L, n), lambda i: (i, 0, 0))
    pair2 = lambda s0, s1: pl.BlockSpec((1, s0, s1), lambda i: (i % 2, 0, 0))

    xt, gz, delta, bmt, cmt, wt = pl.pallas_call(
        _k1_body,
        grid=(2 * B,),
        in_specs=[
            big(DM),
            pl.BlockSpec((2 * DI, DM), lambda i: (0, 0)),
            pair2(DC, DI), pair2(1, DI), pair2(R, DI), pair2(DS, DI),
            pair2(DS, DI), pair2(DI, R), pair2(1, DI),
            pl.BlockSpec((DI, DI), lambda i: (0, 0)),
            pl.BlockSpec((1, 1, DI), lambda i: (0, 0, 0)),
            pl.BlockSpec((NE, DI), lambda i: (0, 0)),
            pl.BlockSpec((NE, 1), lambda i: (0, 0)),
        ],
        out_specs=[
            big(DI), big(DI), big(DI),
            pl.BlockSpec((1, DS, L), lambda i: (i, 0, 0)),
            pl.BlockSpec((1, DS, L), lambda i: (i, 0, 0)),
            pl.BlockSpec((1, NE, L), lambda i: (i // 2, 0, 0)),
        ],
        out_shape=[
            jax.ShapeDtypeStruct((2 * B, L, DI), F32),
            jax.ShapeDtypeStruct((2 * B, L, DI), F32),
            jax.ShapeDtypeStruct((2 * B, L, DI), F32),
            jax.ShapeDtypeStruct((2 * B, DS, L), F32),
            jax.ShapeDtypeStruct((2 * B, DS, L), F32),
            jax.ShapeDtypeStruct((B, NE, L), F32),
        ],
        compiler_params=pltpu.CompilerParams(
            dimension_semantics=("parallel",),
            vmem_limit_bytes=_VMEM_LIM),
    )(f32(hid2), f32(in_proj_w), f32(convw2), f32(convb2), f32(wdt2),
      f32(wb2), f32(wc2), f32(dtp2), f32(dtb2), f32(mlp_w1), f32(b1r),
      f32(mlp_w2), f32(b2r))

    y = pl.pallas_call(
        _k2_body,
        grid=(2 * B,),
        in_specs=[
            big(DI), big(DI), big(DI),
            pl.BlockSpec((1, DS, L), lambda i: (i, 0, 0)),
            pl.BlockSpec((1, DS, L), lambda i: (i, 0, 0)),
            pl.BlockSpec((1, NE, L), lambda i: (i // 2, 0, 0)),
            pair2(DS, DI),
            pair2(1, DI),
        ],
        out_specs=big(DI),
        out_shape=jax.ShapeDtypeStruct((2 * B, L, DI), F32),
        compiler_params=pltpu.CompilerParams(
            dimension_semantics=("parallel",),
            vmem_limit_bytes=_VMEM_LIM),
    )(delta, xt, gz, bmt, cmt, wt, f32(alogt2), f32(d2))

    yf = y[0::2]
    yb = y[1::2][:, ::-1, :]

    out = pl.pallas_call(
        _k3_body,
        grid=(B,),
        in_specs=[
            big(DI), big(DI),
            pl.BlockSpec((DM, DI), lambda i: (0, 0)),
        ],
        out_specs=big(DM),
        out_shape=jax.ShapeDtypeStruct((B, L, DM), F32),
        compiler_params=pltpu.CompilerParams(
            dimension_semantics=("parallel",),
            vmem_limit_bytes=_VMEM_LIM),
    )(yf, yb, f32(out_proj_w))
    return out


# speedup vs baseline: 11.0902x; 1.3763x over previous
import jax
import jax.numpy as jnp
from jax import lax
from jax.experimental import pallas as pl
from jax.experimental.pallas import tpu as pltpu

DM = 768
DS = 16
DC = 4
DI = 1536
NE = 4
DE = 4
R = 48
B = 4
L = 512
TC = 128
F32 = jnp.float32
_VMEM_LIM = 64 * 1024 * 1024


def _dot_nt(a, b):
    return lax.dot_general(a, b, (((1,), (1,)), ((), ())),
                           preferred_element_type=F32)


def _body(hid_ref, win_ref, convw_ref, convb_ref, wdt_ref, wb_ref, wc_ref,
          dtp_ref, dtb_ref, w1_ref, b1_ref, w2_ref, b2_ref, alogt_ref,
          d_ref, wout_ref, out_ref,
          xt_s, gz_s, delta_s, bmt_s, cmt_s, wt_s, yacc_s):
    i = pl.program_id(0)
    is_fwd = i % 2 == 0
    hid = hid_ref[0]
    xz = _dot_nt(hid, win_ref[...])
    x = xz[:, :DI]
    z = xz[:, DI:]

    wt = convw_ref[0]
    acc = x * wt[DC - 1:DC, :]
    for k in range(DC - 1):
        shift = DC - 1 - k
        xk = jnp.concatenate(
            [jnp.zeros((shift, DI), F32), x[:L - shift, :]], axis=0)
        acc = acc + xk * wt[k:k + 1, :]
    acc = acc + convb_ref[0]
    xc = acc * jax.nn.sigmoid(acc)

    dtr = _dot_nt(xc, wdt_ref[0])
    dv = _dot_nt(dtr, dtp_ref[0]) + dtb_ref[0]
    xt_s[...] = xc
    gz_s[...] = z * jax.nn.sigmoid(z)
    delta_s[...] = jnp.maximum(dv, 0.0) + jnp.log1p(jnp.exp(-jnp.abs(dv)))
    bmt_s[...] = _dot_nt(wb_ref[0], xc)
    cmt_s[...] = _dot_nt(wc_ref[0], xc)

    @pl.when(is_fwd)
    def _router():
        h1 = _dot_nt(xc, w1_ref[...]) + b1_ref[0]
        h1 = h1 * jax.nn.sigmoid(h1)
        gt = _dot_nt(w2_ref[...], h1) + b2_ref[...]
        gt = jax.nn.sigmoid(gt)
        m = jnp.max(gt, axis=0, keepdims=True)
        e = jnp.exp(gt - m)
        wt_s[0:NE, :] = e / jnp.sum(e, axis=0, keepdims=True)

    a_t = -jnp.exp(alogt_ref[0])
    drow_skip = d_ref[0]

    def chunk(c, h):
        base = c * TC
        sl = pl.ds(base, TC)
        delta_c = delta_s[sl, :]
        xt_c = xt_s[sl, :]
        gz_c = gz_s[sl, :]
        bmt_c = bmt_s[:, sl]
        cwt = cmt_s[:, sl] * jnp.concatenate(
            [wt_s[e:e + 1, sl] for e in range(NE) for _ in range(DE)],
            axis=0)
        rows = []
        for t in range(TC):
            drow = delta_c[t:t + 1, :]
            dxrow = drow * xt_c[t:t + 1, :]
            da = jnp.exp(a_t * drow)
            h = da * h + dxrow * bmt_c[:, t:t + 1]
            rows.append(jnp.sum(h * cwt[:, t:t + 1], axis=0, keepdims=True))
        y_g = (jnp.concatenate(rows, axis=0)
               + xt_c * drow_skip) * gz_c

        @pl.when(is_fwd)
        def _store_fwd():
            yacc_s[sl, :] = y_g

        @pl.when(jnp.logical_not(is_fwd))
        def _store_bwd():
            y_rev = jnp.concatenate(
                [y_g[t:t + 1, :] for t in range(TC - 1, -1, -1)], axis=0)
            fsl = pl.ds(L - base - TC, TC)
            yacc_s[fsl, :] = yacc_s[fsl, :] + y_rev
        return h

    lax.fori_loop(0, L // TC, chunk, jnp.zeros((DS, DI), F32))

    @pl.when(jnp.logical_not(is_fwd))
    def _out():
        out_ref[0] = _dot_nt(yacc_s[...], wout_ref[...])


def kernel(hidden_states, in_proj_w, conv_w, conv_b, conv_w_b, conv_b_b,
           x_proj_w, x_proj_w_bwd, dt_proj_w, dt_proj_bias, dt_proj_w_bwd,
           dt_proj_bias_bwd, A_log, A_b_log, D, D_b, mlp_w1, mlp_b1,
           mlp_w2, mlp_b2, out_proj_w):
    f32 = lambda v: v.astype(F32)
    hid2 = jnp.stack([hidden_states, hidden_states[:, ::-1, :]],
                     axis=1).reshape(2 * B, L, DM)
    convw2 = jnp.stack([conv_w[:, 0, :].T, conv_w_b[:, 0, :].T])
    convb2 = jnp.stack([conv_b, conv_b_b])[:, None, :]
    wdt2 = jnp.stack([x_proj_w[:R], x_proj_w_bwd[:R]])
    wb2 = jnp.stack([x_proj_w[R:R + DS], x_proj_w_bwd[R:R + DS]])
    wc2 = jnp.stack([x_proj_w[R + DS:], x_proj_w_bwd[R + DS:]])
    dtp2 = jnp.stack([dt_proj_w, dt_proj_w_bwd])
    dtb2 = jnp.stack([dt_proj_bias, dt_proj_bias_bwd])[:, None, :]
    alogt2 = jnp.stack([A_log.T, A_b_log.T])
    d2 = jnp.stack([D, D_b])[:, None, :]
    b1r = mlp_b1[None, None, :]
    b2r = mlp_b2[:, None]

    pair2 = lambda s0, s1: pl.BlockSpec((1, s0, s1), lambda i: (i % 2, 0, 0))

    out = pl.pallas_call(
        _body,
        grid=(2 * B,),
        in_specs=[
            pl.BlockSpec((1, L, DM), lambda i: (i, 0, 0)),
            pl.BlockSpec((2 * DI, DM), lambda i: (0, 0)),
            pair2(DC, DI), pair2(1, DI), pair2(R, DI), pair2(DS, DI),
            pair2(DS, DI), pair2(DI, R), pair2(1, DI),
            pl.BlockSpec((DI, DI), lambda i: (0, 0)),
            pl.BlockSpec((1, 1, DI), lambda i: (0, 0, 0)),
            pl.BlockSpec((NE, DI), lambda i: (0, 0)),
            pl.BlockSpec((NE, 1), lambda i: (0, 0)),
            pair2(DS, DI),
            pair2(1, DI),
            pl.BlockSpec((DM, DI), lambda i: (0, 0)),
        ],
        out_specs=pl.BlockSpec((1, L, DM), lambda i: (i // 2, 0, 0)),
        out_shape=jax.ShapeDtypeStruct((B, L, DM), F32),
        scratch_shapes=[
            pltpu.VMEM((L, DI), F32),
            pltpu.VMEM((L, DI), F32),
            pltpu.VMEM((L, DI), F32),
            pltpu.VMEM((DS, L), F32),
            pltpu.VMEM((DS, L), F32),
            pltpu.VMEM((8, L), F32),
            pltpu.VMEM((L, DI), F32),
        ],
        compiler_params=pltpu.CompilerParams(
            dimension_semantics=("parallel",),
            vmem_limit_bytes=_VMEM_LIM),
    )(f32(hid2), f32(in_proj_w), f32(convw2), f32(convb2), f32(wdt2),
      f32(wb2), f32(wc2), f32(dtp2), f32(dtb2), f32(mlp_w1), f32(b1r),
      f32(mlp_w2), f32(b2r), f32(alogt2), f32(d2), f32(out_proj_w))
    return out
